# Initial kernel scaffold; baseline (speedup 1.0000x reference)
import jax
import jax.numpy as jnp
from jax.experimental import pallas as pl


def kernel(inputs, emb_weight):
    raise NotImplementedError("write your pallas kernel here")


# speedup vs baseline: 1.1971x; 1.1971x over previous
import functools

import jax
import jax.numpy as jnp
from jax import lax
from jax.experimental import pallas as pl
from jax.experimental.pallas import tpu as pltpu
from jax.experimental.pallas import tpu_sc as plsc

NUM_CODES = 8192
DIM = 256
COMMIT = 0.25
BT = 256
NUM_TOKENS = 8192


def _vq_body(x_ref, e_ref, xsq_ref, esq_ref, idx_ref, dsum_ref):
    t = pl.program_id(0)
    mm = lax.dot_general(
        x_ref[...], e_ref[...], (((1,), (1,)), ((), ())),
        preferred_element_type=jnp.float32)
    dist = (xsq_ref[...] + esq_ref[...]) - 2.0 * mm
    rowmin = jnp.min(dist, axis=1, keepdims=True)
    lanes = lax.broadcasted_iota(jnp.int32, dist.shape, 1)
    idx = jnp.min(jnp.where(dist == rowmin, lanes, NUM_CODES),
                  axis=1, keepdims=True)
    idx_ref[...] = idx
    partial = jnp.sum(rowmin)

    @pl.when(t == 0)
    def _():
        dsum_ref[0, 0] = partial

    @pl.when(t != 0)
    def _():
        dsum_ref[0, 0] += partial


_vq_call = pl.pallas_call(
    _vq_body,
    grid=(NUM_TOKENS // BT,),
    in_specs=[
        pl.BlockSpec((BT, DIM), lambda t: (t, 0)),
        pl.BlockSpec((NUM_CODES, DIM), lambda t: (0, 0)),
        pl.BlockSpec((BT, 1), lambda t: (t, 0)),
        pl.BlockSpec((1, NUM_CODES), lambda t: (0, 0)),
    ],
    out_specs=[
        pl.BlockSpec((BT, 1), lambda t: (t, 0)),
        pl.BlockSpec(memory_space=pltpu.SMEM, block_shape=(1, 1),
                     index_map=lambda t: (0, 0)),
    ],
    out_shape=[
        jax.ShapeDtypeStruct((NUM_TOKENS, 1), jnp.int32),
        jax.ShapeDtypeStruct((1, 1), jnp.float32),
    ],
)


_NW = 32
_BPW = NUM_TOKENS // _NW

@functools.lru_cache(maxsize=1)
def _sc_gather_fn():
    mesh = plsc.VectorSubcoreMesh(
        core_axis_name="c", subcore_axis_name="s",
        num_cores=2, num_subcores=16)

    @functools.partial(
        pl.kernel,
        out_type=jax.ShapeDtypeStruct((NUM_TOKENS, DIM), jnp.float32),
        mesh=mesh,
        scratch_types=[
            pltpu.VMEM((_BPW,), jnp.int32),
            pltpu.VMEM((_BPW, DIM), jnp.float32),
            pltpu.SemaphoreType.DMA,
        ],
    )
    def _sc_gather(table_hbm, idx_hbm, out_hbm, idx_v, rows_v, sem):
        wid = lax.axis_index("s") * 2 + lax.axis_index("c")
        base = wid * _BPW
        pltpu.sync_copy(idx_hbm.at[pl.ds(base, _BPW)], idx_v)
        pltpu.async_copy(table_hbm.at[idx_v], rows_v, sem).wait()
        pltpu.sync_copy(rows_v, out_hbm.at[pl.ds(base, _BPW)])

    return _sc_gather


def kernel(inputs, emb_weight):
    B, C, H, W = inputs.shape
    flat = jnp.transpose(inputs, (0, 2, 3, 1)).reshape(-1, DIM)
    xsq = jnp.sum(flat ** 2, axis=1, keepdims=True)
    esq = jnp.sum(emb_weight ** 2, axis=1)
    idx2, dsum = _vq_call(flat, emb_weight, xsq, esq.reshape(1, NUM_CODES))
    rows = _sc_gather_fn()(emb_weight, idx2.reshape(NUM_TOKENS))
    quantized = jnp.transpose(rows.reshape(B, H, W, C), (0, 3, 1, 2))
    loss = dsum[0, 0] * ((1.0 + COMMIT) / inputs.size)
    return (quantized, loss, idx2)
